# baseline (speedup 1.0000x reference)
import functools
import math

import jax
import jax.numpy as jnp
from jax import lax
from jax.experimental import pallas as pl
from jax.experimental.pallas import tpu as pltpu
from jax.experimental.pallas import tpu_sc as plsc

D = 1024
ROWS = 2 * 4096 - 1
N = ROWS * D
BATCH = 4
NC = 2
CHUNK = 512 * D
NCHUNK_TOTAL = (N + CHUNK - 1) // CHUNK
CHUNKS_PER_SC = NCHUNK_TOTAL // NC
NBUF = 3
LOOKAHEAD = 2


def _sc_body(w_hbm, out_hbm, sb0, sb1, sb2, si0, si1, si2, so0, so1, so2):
    c = lax.axis_index("c")
    s = lax.axis_index("s")
    bufs = [sb0, sb1, sb2]
    sem_in = [si0, si1, si2]
    sem_out = [so0, so1, so2]

    @pl.when(s == 0)
    def _driver():
        def base(k):
            return lax.min((c * CHUNKS_PER_SC + k) * CHUNK, N - CHUNK)

        def issue_gather(k):
            return pltpu.async_copy(
                w_hbm.at[pl.ds(base(k), CHUNK)], bufs[k % NBUF],
                sem_in[k % NBUF])

        def issue_scatters(k):
            return [
                pltpu.async_copy(
                    bufs[k % NBUF],
                    out_hbm.at[pl.ds(b * N + base(k), CHUNK)],
                    sem_out[k % NBUF])
                for b in range(BATCH)
            ]

        gathers = {k: issue_gather(k) for k in range(LOOKAHEAD)}
        scatters = {}
        for g in range(CHUNKS_PER_SC):
            if g - LOOKAHEAD in scatters:
                for h in scatters.pop(g - LOOKAHEAD):
                    h.wait()
            if g + LOOKAHEAD < CHUNKS_PER_SC:
                gathers[g + LOOKAHEAD] = issue_gather(g + LOOKAHEAD)
            gathers.pop(g).wait()
            scatters[g] = issue_scatters(g)

        for hs in scatters.values():
            for h in hs:
                h.wait()


def _sc_embed(w_flat):
    mesh = plsc.VectorSubcoreMesh(core_axis_name="c", subcore_axis_name="s")
    f = functools.partial(
        pl.kernel,
        mesh=mesh,
        out_type=jax.ShapeDtypeStruct((BATCH * N,), jnp.float32),
        scratch_types=(
            [pltpu.VMEM_SHARED((CHUNK,), jnp.float32) for _ in range(NBUF)]
            + [pltpu.SemaphoreType.DMA for _ in range(2 * NBUF)]
        ),
    )(_sc_body)
    return f(w_flat)


def kernel(input, weights):
    del input
    out_flat = _sc_embed(weights.reshape(N))
    return out_flat.reshape(BATCH, ROWS, D)


# speedup vs baseline: 4.2545x; 4.2545x over previous
import functools
import math

import jax
import jax.numpy as jnp
from jax.experimental import pallas as pl
from jax.experimental.pallas import tpu as pltpu

D = 1024
ROWS = 2 * 4096 - 1
BATCH = 4
BLOCK_ROWS = 256
GRID = (ROWS + BLOCK_ROWS - 1) // BLOCK_ROWS
SCALE = math.sqrt(D)


def _body(w_ref, o_ref):
    scaled = w_ref[...] * SCALE
    o_ref[...] = jnp.broadcast_to(scaled[None, :, :], (BATCH,) + scaled.shape)


def _tc_embed(weights):
    return pl.pallas_call(
        _body,
        grid=(GRID,),
        in_specs=[pl.BlockSpec((BLOCK_ROWS, D), lambda i: (i, 0))],
        out_specs=pl.BlockSpec((BATCH, BLOCK_ROWS, D), lambda i: (0, i, 0)),
        out_shape=jax.ShapeDtypeStruct((BATCH, ROWS, D), jnp.float32),
        compiler_params=pltpu.CompilerParams(
            dimension_semantics=("arbitrary",),
        ),
    )(weights)


def kernel(input, weights):
    del input
    return _tc_embed(weights)
